# baseline (speedup 1.0000x reference)
import functools

import jax
import jax.numpy as jnp
from jax import lax
from jax.experimental import pallas as pl
from jax.experimental.pallas import tpu as pltpu
from jax.experimental.pallas import tpu_sc as plsc

NUM_ENTITIES = 1000000
NUM_RELATIONS = 1000
EMBED_DIM = 64
BATCH = 16384

_NC = 2
_NS = 16
_NW = _NC * _NS
_BPW = BATCH // _NW
_L = 16


def _tec_body(heads_hbm, relations_hbm, tails_hbm, entity_hbm, relation_hbm,
              out_hbm, h_idx, r_idx, t_idx, h_rows, r_rows, t_rows, out_v,
              sem):
    wid = lax.axis_index("s") * _NC + lax.axis_index("c")
    base = wid * _BPW

    pltpu.sync_copy(heads_hbm.at[pl.ds(base, _BPW)], h_idx)
    pltpu.sync_copy(relations_hbm.at[pl.ds(base, _BPW)], r_idx)
    pltpu.sync_copy(tails_hbm.at[pl.ds(base, _BPW)], t_idx)

    c1 = pltpu.make_async_copy(entity_hbm.at[h_idx], h_rows, sem)
    c2 = pltpu.make_async_copy(relation_hbm.at[r_idx], r_rows, sem)
    c3 = pltpu.make_async_copy(entity_hbm.at[t_idx], t_rows, sem)
    c1.start()
    c2.start()
    c3.start()
    c1.wait()
    c2.wait()
    c3.wait()

    lanes = lax.iota(jnp.int32, _L)

    def _sqrt16(x):
        i = plsc.bitcast(x, jnp.int32)
        i = jnp.int32(0x5F3759DF) - lax.shift_right_logical(i, 1)
        y = plsc.bitcast(i, jnp.float32)
        xh = x * jnp.float32(0.5)
        for _ in range(3):
            y = y * (jnp.float32(1.5) - xh * y * y)
        return jnp.where(x > 0, x * y, jnp.float32(0.0))

    def step(i, carry):
        rows16 = i * _L + lanes
        acc = jnp.zeros((_L,), jnp.float32)
        for d in range(EMBED_DIM):
            col = jnp.full((_L,), d, jnp.int32)
            hv = plsc.load_gather(h_rows, [rows16, col])
            rv = plsc.load_gather(r_rows, [rows16, col])
            tv = plsc.load_gather(t_rows, [rows16, col])
            dv = (hv + rv) - tv
            acc = acc + dv * dv
        out_v[pl.ds(i * _L, _L)] = _sqrt16(acc)
        return carry

    lax.fori_loop(0, _BPW // _L, step, 0)

    pltpu.sync_copy(out_v, out_hbm.at[pl.ds(base, _BPW)])


@jax.jit
def _transe_sc(heads, relations, tails, entity_table, relation_table):
    mesh = plsc.VectorSubcoreMesh(core_axis_name="c", subcore_axis_name="s")
    kfn = functools.partial(
        pl.kernel,
        mesh=mesh,
        compiler_params=pltpu.CompilerParams(
            needs_layout_passes=False, use_tc_tiling_on_sc=False),
        out_type=jax.ShapeDtypeStruct((BATCH,), jnp.float32),
        scratch_types=[
            pltpu.VMEM((_BPW,), jnp.int32),
            pltpu.VMEM((_BPW,), jnp.int32),
            pltpu.VMEM((_BPW,), jnp.int32),
            pltpu.VMEM((_BPW, EMBED_DIM), jnp.float32),
            pltpu.VMEM((_BPW, EMBED_DIM), jnp.float32),
            pltpu.VMEM((_BPW, EMBED_DIM), jnp.float32),
            pltpu.VMEM((_BPW,), jnp.float32),
            pltpu.SemaphoreType.DMA,
        ],
    )(_tec_body)
    return kfn(heads, relations, tails, entity_table, relation_table)


def kernel(heads, relations, tails, entity_table, relation_table):
    heads = jnp.asarray(heads, jnp.int32)
    relations = jnp.asarray(relations, jnp.int32)
    tails = jnp.asarray(tails, jnp.int32)
    return _transe_sc(heads, relations, tails, entity_table, relation_table)


# speedup vs baseline: 1.6247x; 1.6247x over previous
import functools

import jax
import jax.numpy as jnp
from jax import lax
from jax.experimental import pallas as pl
from jax.experimental.pallas import tpu as pltpu
from jax.experimental.pallas import tpu_sc as plsc

NUM_ENTITIES = 1000000
NUM_RELATIONS = 1000
EMBED_DIM = 64
BATCH = 16384

_NC = 2
_NS = 16
_NW = _NC * _NS
_BPW = BATCH // _NW
_L = 16
_CH = 16
_NCH = _BPW // _CH


def _tec_body(heads_hbm, relations_hbm, tails_hbm, entity_hbm, relation_hbm,
              out_hbm, h_idx, r_idx, t_idx, h_rows, r_rows, t_rows, out_v,
              sem0, sem1):
    wid = lax.axis_index("s") * _NC + lax.axis_index("c")
    base = wid * _BPW

    pltpu.sync_copy(heads_hbm.at[pl.ds(base, _BPW)], h_idx)
    pltpu.sync_copy(relations_hbm.at[pl.ds(base, _BPW)], r_idx)
    pltpu.sync_copy(tails_hbm.at[pl.ds(base, _BPW)], t_idx)

    sems = (sem0, sem1)

    def _row_copies(c, buf):
        hvec = h_idx[pl.ds(c * _CH, _CH)]
        rvec = r_idx[pl.ds(c * _CH, _CH)]
        tvec = t_idx[pl.ds(c * _CH, _CH)]
        cps = []
        for j in range(_CH):
            dst = buf * _CH + j
            cps.append(pltpu.make_async_copy(
                entity_hbm.at[pl.ds(hvec[j], 1), :],
                h_rows.at[pl.ds(dst, 1), :],
                sems[buf]))
            cps.append(pltpu.make_async_copy(
                relation_hbm.at[pl.ds(rvec[j], 1), :],
                r_rows.at[pl.ds(dst, 1), :],
                sems[buf]))
            cps.append(pltpu.make_async_copy(
                entity_hbm.at[pl.ds(tvec[j], 1), :],
                t_rows.at[pl.ds(dst, 1), :],
                sems[buf]))
        return cps

    def _fire(c, buf):
        for cp in _row_copies(c, buf):
            cp.start()

    def _drain(c, buf):
        for cp in _row_copies(c, buf):
            cp.wait()

    lanes = lax.iota(jnp.int32, _L)

    def _sqrt16(x):
        i = plsc.bitcast(x, jnp.int32)
        i = jnp.int32(0x5F3759DF) - lax.shift_right_logical(i, 1)
        y = plsc.bitcast(i, jnp.float32)
        xh = x * jnp.float32(0.5)
        for _ in range(3):
            y = y * (jnp.float32(1.5) - xh * y * y)
        return jnp.where(x > 0, x * y, jnp.float32(0.0))

    def _compute(c, buf):
        rows16 = jnp.int32(buf * _CH) + lanes
        acc = jnp.zeros((_L,), jnp.float32)
        for d in range(EMBED_DIM):
            col = jnp.full((_L,), d, jnp.int32)
            hv = plsc.load_gather(h_rows, [rows16, col])
            rv = plsc.load_gather(r_rows, [rows16, col])
            tv = plsc.load_gather(t_rows, [rows16, col])
            dv = (hv + rv) - tv
            acc = acc + dv * dv
        out_v[pl.ds(c * _CH, _CH)] = _sqrt16(acc)

    _fire(0, 0)

    def step(cc, carry):
        c0 = cc * 2
        c1 = c0 + 1
        _fire(c1, 1)
        _drain(c0, 0)
        _compute(c0, 0)

        @pl.when(c1 + 1 < _NCH)
        def _():
            _fire(c1 + 1, 0)

        _drain(c1, 1)
        _compute(c1, 1)
        return carry

    lax.fori_loop(0, _NCH // 2, step, 0)

    pltpu.sync_copy(out_v, out_hbm.at[pl.ds(base, _BPW)])


@jax.jit
def _transe_sc(heads, relations, tails, entity_table, relation_table):
    mesh = plsc.VectorSubcoreMesh(core_axis_name="c", subcore_axis_name="s")
    kfn = functools.partial(
        pl.kernel,
        mesh=mesh,
        compiler_params=pltpu.CompilerParams(
            needs_layout_passes=False, use_tc_tiling_on_sc=True),
        out_type=jax.ShapeDtypeStruct((BATCH,), jnp.float32),
        scratch_types=[
            pltpu.VMEM((_BPW,), jnp.int32),
            pltpu.VMEM((_BPW,), jnp.int32),
            pltpu.VMEM((_BPW,), jnp.int32),
            pltpu.VMEM((2 * _CH, EMBED_DIM), jnp.float32),
            pltpu.VMEM((2 * _CH, EMBED_DIM), jnp.float32),
            pltpu.VMEM((2 * _CH, EMBED_DIM), jnp.float32),
            pltpu.VMEM((_BPW,), jnp.float32),
            pltpu.SemaphoreType.DMA,
            pltpu.SemaphoreType.DMA,
        ],
    )(_tec_body)
    return kfn(heads, relations, tails, entity_table, relation_table)


def kernel(heads, relations, tails, entity_table, relation_table):
    heads = jnp.asarray(heads, jnp.int32)
    relations = jnp.asarray(relations, jnp.int32)
    tails = jnp.asarray(tails, jnp.int32)
    return _transe_sc(heads, relations, tails, entity_table, relation_table)
